# baseline (speedup 1.0000x reference)
import jax
import jax.numpy as jnp
from jax import lax
from jax.experimental import pallas as pl
from jax.experimental.pallas import tpu as pltpu
from jax.experimental.pallas import tpu_sc as plsc

N = 10000
E = 320000
NPAD = 10240
NSC = 2
NSUB = 16
CHR = 32
NIT = 324
EP = NSC * NSUB * NIT * CHR
NBUF = 6
LEAD = 3
RPT = NPAD // NSUB
EPS = E // (NSC * NSUB)
BN = 1024


def _sc_mesh():
    return plsc.VectorSubcoreMesh(core_axis_name="c", subcore_axis_name="s")


def _deg_body(dst_hbm, out_hbm, dbuf, part):
    c = lax.axis_index("c")
    s = lax.axis_index("s")
    wid = c * NSUB + s
    pltpu.sync_copy(dst_hbm.at[wid], dbuf)
    z16 = jnp.zeros((16,), jnp.float32)

    def zero_body(i, _):
        part[pl.ds(i * 16, 16)] = z16
        return 0

    lax.fori_loop(0, NPAD // 16, zero_body, 0)
    ones = jnp.ones((16,), jnp.float32)

    def cnt_body(i, _):
        iv = dbuf[pl.ds(i * 16, 16)]
        plsc.addupdate_scatter(part, [iv], ones)
        return 0

    lax.fori_loop(0, EPS // 16, cnt_body, 0)
    pltpu.sync_copy(part, out_hbm.at[wid])


def _degree_counts(dst_r):
    return pl.kernel(
        _deg_body,
        out_type=jax.ShapeDtypeStruct((NSC * NSUB, NPAD), jnp.float32),
        mesh=_sc_mesh(),
        compiler_params=pltpu.CompilerParams(needs_layout_passes=False),
        scratch_types=[
            pltpu.VMEM((EPS,), jnp.int32),
            pltpu.VMEM((NPAD,), jnp.float32),
        ],
    )(dst_r)


def _agg_body(xs_hbm, srcr_hbm, dstr_hbm, out_hbm, didx, sidx, zbuf, acc, *ring):
    rows = ring[:NBUF]
    gsems = ring[NBUF:2 * NBUF]
    ssems = ring[2 * NBUF:]
    c = lax.axis_index("c")
    s = lax.axis_index("s")
    pltpu.sync_copy(srcr_hbm.at[c, s], sidx)
    pltpu.sync_copy(dstr_hbm.at[c, s], didx)
    z16 = jnp.zeros((16,), jnp.float32)

    def zrow(i, _):
        def zcol(j, _):
            zbuf[i, pl.ds(j * 16, 16)] = z16
            return 0

        lax.fori_loop(0, 128 // 16, zcol, 0)
        return 0

    lax.fori_loop(0, 16, zrow, 0)

    def zacc(k, _):
        pltpu.sync_copy(zbuf, acc.at[pl.ds(s * RPT + k * 16, 16)])
        return 0

    lax.fori_loop(0, RPT // 16, zacc, 0)
    plsc.subcore_barrier()

    def grp(p, _):
        base = p * NBUF
        for b in range(NBUF):
            pltpu.async_copy(xs_hbm.at[sidx.at[base + b]], rows[b], gsems[b])
        for b in range(NBUF):
            pltpu.make_async_copy(
                xs_hbm.at[sidx.at[base + b]], rows[b], gsems[b]).wait()
        for b in range(NBUF):
            pltpu.async_copy(rows[b], acc.at[didx.at[base + b]], ssems[b],
                             add=True)
        for b in range(NBUF):
            pltpu.make_async_copy(
                rows[b], acc.at[didx.at[base + b]], ssems[b]).wait()
        return 0

    lax.fori_loop(0, NIT // NBUF, grp, 0)
    plsc.subcore_barrier()
    pltpu.sync_copy(acc.at[pl.ds(s * RPT, RPT)],
                    out_hbm.at[c, pl.ds(s * RPT, RPT)])


def _aggregate(xs, srcr, dstr):
    return pl.kernel(
        _agg_body,
        out_type=jax.ShapeDtypeStruct((NSC, NPAD, 128), jnp.float32),
        mesh=_sc_mesh(),
        compiler_params=pltpu.CompilerParams(
            needs_layout_passes=False, use_tc_tiling_on_sc=False),
        scratch_types=(
            [pltpu.VMEM((NIT, CHR), jnp.int32),
             pltpu.VMEM((NIT, CHR), jnp.int32),
             pltpu.VMEM((16, 128), jnp.float32),
             pltpu.VMEM_SHARED((NPAD, 128), jnp.float32)]
            + [pltpu.VMEM((CHR, 128), jnp.float32) for _ in range(NBUF)]
            + [pltpu.SemaphoreType.DMA for _ in range(2 * NBUF)]
        ),
    )(xs, srcr, dstr)


_F32 = jnp.float32
_GRID = NPAD // BN


def _dot(a, b):
    return jnp.dot(a, b, preferred_element_type=_F32)


def _kdeg(p_ref, o_ref):
    o_ref[...] = lax.rsqrt(jnp.sum(p_ref[...], axis=0) + 1.0)


def _k0(x_ref, d_ref, o_ref):
    o_ref[...] = x_ref[...] * d_ref[...]


def _k12(a_ref, x_ref, d_ref, w1_ref, b1_ref, w2_ref, oa_ref, ob_ref):
    d = d_ref[...]
    z = (a_ref[0] + a_ref[1] + x_ref[...]) * d
    y1 = jnp.maximum(_dot(z, w1_ref[...]) + b1_ref[...], 0.0)
    oa_ref[...] = _dot(y1, w2_ref[0]) * d
    ob_ref[...] = _dot(y1, w2_ref[1]) * d


def _k23(aa_ref, ab_ref, ha_ref, hb_ref, d_ref, b2_ref, w3_ref, o_ref):
    d = d_ref[...]
    y2a = jnp.maximum((aa_ref[0] + aa_ref[1] + ha_ref[...]) * d + b2_ref[0], 0.0)
    y2b = jnp.maximum((ab_ref[0] + ab_ref[1] + hb_ref[...]) * d + b2_ref[1], 0.0)
    o_ref[...] = (_dot(y2a, w3_ref[0:128, :]) + _dot(y2b, w3_ref[128:256, :])) * d


def _k34(a_ref, h_ref, d_ref, b3_ref, o_ref):
    d = d_ref[...]
    z = (a_ref[0] + a_ref[1] + h_ref[...]) * d
    o_ref[...] = jnp.maximum(z + b3_ref[...], 0.0) * d


def _k4(a_ref, x_ref, d_ref, w4_ref, b4_ref, o_ref):
    i = pl.program_id(0)
    z = (a_ref[0] + a_ref[1] + x_ref[...]) * d_ref[...]
    y4 = jnp.maximum(_dot(z, w4_ref[...]) + b4_ref[...], 0.0)
    row = i * BN + lax.broadcasted_iota(jnp.int32, (BN, 1), 0)
    y4 = jnp.where(row < N, y4, 0.0)
    part = jnp.sum(y4, axis=0, keepdims=True) * (1.0 / N)

    @pl.when(i == 0)
    def _():
        o_ref[...] = part

    @pl.when(i > 0)
    def _():
        o_ref[...] = o_ref[...] + part


def _pair_spec():
    return pl.BlockSpec((2, BN, 128), lambda i: (0, i, 0))


def _col_spec(w):
    return pl.BlockSpec((BN, w), lambda i: (i, 0))


def _full_spec(shape):
    nd = len(shape)
    return pl.BlockSpec(shape, lambda i, _n=nd: (0,) * _n)


def _tc_call(body, in_specs, out_specs, out_shape, acc=False):
    sem = ("arbitrary",) if acc else ("parallel",)
    return pl.pallas_call(
        body,
        grid=(_GRID,),
        in_specs=in_specs,
        out_specs=out_specs,
        out_shape=out_shape,
        compiler_params=pltpu.CompilerParams(dimension_semantics=sem),
    )


def kernel(x, edge_index, edge_attr, W1, b1, W2, b2, W3, b3, W4, b4):
    del edge_attr
    src = edge_index[0]
    dst = edge_index[1]

    degp = _degree_counts(dst.reshape(NSC * NSUB, EPS))
    dinv2d = pl.pallas_call(
        _kdeg,
        out_shape=jax.ShapeDtypeStruct((NPAD // 128, 128), _F32),
    )(degp.reshape(NSC * NSUB, NPAD // 128, 128))
    dinv = dinv2d.reshape(NPAD)
    d128 = jnp.broadcast_to(dinv[:, None], (NPAD, 128))

    srcr = jnp.concatenate(
        [src, jnp.zeros((EP - E,), src.dtype)]).reshape(NSC, NSUB, NIT, CHR)
    dstr = jnp.concatenate(
        [dst, jnp.full((EP - E,), NPAD - 1, dst.dtype)]).reshape(NSC, NSUB, NIT, CHR)

    x_pad = jnp.pad(x, ((0, NPAD - N), (0, 0)))
    b1r = b1.reshape(1, 512)
    w2s = W2.reshape(512, 2, 128).transpose(1, 0, 2)
    b2s = b2.reshape(2, 1, 128)
    b3r = b3.reshape(1, 128)
    b4r = b4.reshape(1, 200)
    tbl = jax.ShapeDtypeStruct((NPAD, 128), _F32)

    xs1 = _tc_call(
        _k0, [_col_spec(128), _col_spec(128)], _col_spec(128), tbl,
    )(x_pad, d128)
    a1 = _aggregate(xs1, srcr, dstr)
    h2a, h2b = _tc_call(
        _k12,
        [_pair_spec(), _col_spec(128), _col_spec(128),
         _full_spec((128, 512)), _full_spec((1, 512)), _full_spec((2, 512, 128))],
        [_col_spec(128), _col_spec(128)],
        [tbl, tbl],
    )(a1, xs1, d128, W1, b1r, w2s)

    a2a = _aggregate(h2a, srcr, dstr)
    a2b = _aggregate(h2b, srcr, dstr)
    h3 = _tc_call(
        _k23,
        [_pair_spec(), _pair_spec(), _col_spec(128), _col_spec(128),
         _col_spec(128), _full_spec((2, 1, 128)), _full_spec((256, 128))],
        _col_spec(128),
        tbl,
    )(a2a, a2b, h2a, h2b, d128, b2s, W3)

    a3 = _aggregate(h3, srcr, dstr)
    xs4 = _tc_call(
        _k34,
        [_pair_spec(), _col_spec(128), _col_spec(128), _full_spec((1, 128))],
        _col_spec(128),
        tbl,
    )(a3, h3, d128, b3r)

    a4 = _aggregate(xs4, srcr, dstr)
    out = _tc_call(
        _k4,
        [_pair_spec(), _col_spec(128), _col_spec(128),
         _full_spec((128, 200)), _full_spec((1, 200))],
        pl.BlockSpec((1, 200), lambda i: (0, 0)),
        jax.ShapeDtypeStruct((1, 200), _F32),
        acc=True,
    )(a4, xs4, d128, W4, b4r)
    return out


# speedup vs baseline: 2.2924x; 2.2924x over previous
import functools

import jax
import jax.numpy as jnp
from jax import lax
from jax.experimental import pallas as pl
from jax.experimental.pallas import tpu as pltpu
from jax.experimental.pallas import tpu_sc as plsc

N = 10000
E = 320000
NPAD = 10240
NSC = 2
NSUB = 16
CH = 128
EP = 327680
RPT = NPAD // NSUB
EPS = E // (NSC * NSUB)
NIT = EP // (NSUB * CH)
NP = NIT // 2
BN = 1024


def _sc_mesh():
    return plsc.VectorSubcoreMesh(core_axis_name="c", subcore_axis_name="s")


def _deg_body(dst_hbm, out_hbm, dbuf, part):
    c = lax.axis_index("c")
    s = lax.axis_index("s")
    wid = c * NSUB + s
    pltpu.sync_copy(dst_hbm.at[wid], dbuf)
    z16 = jnp.zeros((16,), jnp.float32)

    def zero_body(i, _):
        part[pl.ds(i * 16, 16)] = z16
        return 0

    lax.fori_loop(0, NPAD // 16, zero_body, 0)
    ones = jnp.ones((16,), jnp.float32)

    def cnt_body(i, _):
        iv = dbuf[pl.ds(i * 16, 16)]
        plsc.addupdate_scatter(part, [iv], ones)
        return 0

    lax.fori_loop(0, EPS // 16, cnt_body, 0)
    pltpu.sync_copy(part, out_hbm.at[wid])


def _degree_counts(dst_r):
    return pl.kernel(
        _deg_body,
        out_type=jax.ShapeDtypeStruct((NSC * NSUB, NPAD), jnp.float32),
        mesh=_sc_mesh(),
        compiler_params=pltpu.CompilerParams(needs_layout_passes=False),
        scratch_types=[
            pltpu.VMEM((EPS,), jnp.int32),
            pltpu.VMEM((NPAD,), jnp.float32),
        ],
    )(dst_r)


DH = 64


NB = 4
NG = NIT // NB


def _agg_body(xs_hbm, srcr2_hbm, dstr_hbm, out_hbm,
              didx, sidx, zbuf, acc, *ring):
    rows = ring[:NB]
    sems = ring[NB:]
    c = lax.axis_index("c")
    s = lax.axis_index("s")
    pltpu.sync_copy(srcr2_hbm.at[c, s], sidx)
    pltpu.sync_copy(dstr_hbm.at[s], didx)
    z16 = jnp.zeros((16,), jnp.float32)

    def zrow(i, _):
        def zcol(j, _):
            zbuf[i, pl.ds(j * 16, 16)] = z16
            return 0

        lax.fori_loop(0, DH // 16, zcol, 0)
        return 0

    lax.fori_loop(0, CH, zrow, 0)

    def zacc(k, _):
        pltpu.sync_copy(zbuf, acc.at[pl.ds(s * RPT + k * CH, CH)])
        return 0

    lax.fori_loop(0, RPT // CH, zacc, 0)
    plsc.subcore_barrier()

    for b in range(NB):
        pltpu.async_copy(xs_hbm.at[sidx.at[b]], rows[b], sems[b])

    def _slot(i, b):
        pltpu.make_async_copy(xs_hbm.at[sidx.at[i]], rows[b], sems[b]).wait()
        pltpu.sync_copy(rows[b], acc.at[didx.at[i]], add=True)

        @pl.when(i + NB < NIT)
        def _():
            pltpu.async_copy(xs_hbm.at[sidx.at[i + NB]], rows[b], sems[b])

    def grp(p, _):
        base = p * NB
        for b in range(NB):
            _slot(base + b, b)
        return 0

    lax.fori_loop(0, NG, grp, 0)
    plsc.subcore_barrier()
    pltpu.sync_copy(acc.at[pl.ds(s * RPT, RPT)],
                    out_hbm.at[pl.ds(c * NPAD + s * RPT, RPT)])


def _aggregate(xs, srcr2, dstr):
    return pl.kernel(
        _agg_body,
        out_type=jax.ShapeDtypeStruct((2 * NPAD, DH), jnp.float32),
        mesh=_sc_mesh(),
        compiler_params=pltpu.CompilerParams(
            needs_layout_passes=False, use_tc_tiling_on_sc=False),
        scratch_types=(
            [pltpu.VMEM((NIT, CH), jnp.int32),
             pltpu.VMEM((NIT, CH), jnp.int32),
             pltpu.VMEM((CH, DH), jnp.float32),
             pltpu.VMEM_SHARED((NPAD, DH), jnp.float32)]
            + [pltpu.VMEM((CH, DH), jnp.float32) for _ in range(NB)]
            + [pltpu.SemaphoreType.DMA for _ in range(NB)]
        ),
    )(xs.reshape(2 * NPAD, DH), srcr2, dstr).reshape(2, NPAD, DH)


_F32 = jnp.float32
_GRID = NPAD // BN


def _dot(a, b):
    return jnp.dot(a, b, preferred_element_type=_F32)


def _kdeg(p_ref, o_ref):
    o_ref[...] = lax.rsqrt(jnp.sum(p_ref[...], axis=0) + 1.0)


def _k0(x_ref, d_ref, o_ref):
    d = d_ref[...]
    o_ref[0] = x_ref[0] * d
    o_ref[1] = x_ref[1] * d


def _k12(a_ref, x_ref, d64_ref, w1_ref, b1_ref, w2_ref, oa_ref, ob_ref):
    d64 = d64_ref[...]
    z0 = (a_ref[0] + x_ref[0]) * d64
    z1 = (a_ref[1] + x_ref[1]) * d64
    y1 = jnp.maximum(
        _dot(z0, w1_ref[0:64, :]) + _dot(z1, w1_ref[64:128, :]) + b1_ref[...],
        0.0)
    oa_ref[0] = _dot(y1, w2_ref[0]) * d64
    oa_ref[1] = _dot(y1, w2_ref[1]) * d64
    ob_ref[0] = _dot(y1, w2_ref[2]) * d64
    ob_ref[1] = _dot(y1, w2_ref[3]) * d64


def _k23(aa_ref, ab_ref, ha_ref, hb_ref, d64_ref, b2_ref, w3_ref, o_ref):
    d64 = d64_ref[...]
    y2 = [
        jnp.maximum((aa_ref[0] + ha_ref[0]) * d64 + b2_ref[0], 0.0),
        jnp.maximum((aa_ref[1] + ha_ref[1]) * d64 + b2_ref[1], 0.0),
        jnp.maximum((ab_ref[0] + hb_ref[0]) * d64 + b2_ref[2], 0.0),
        jnp.maximum((ab_ref[1] + hb_ref[1]) * d64 + b2_ref[3], 0.0),
    ]
    for co in range(2):
        o_ref[co] = sum(_dot(y2[q], w3_ref[q, co]) for q in range(4)) * d64


def _k34(a_ref, h_ref, d64_ref, b3_ref, o_ref):
    d64 = d64_ref[...]
    o_ref[0] = jnp.maximum((a_ref[0] + h_ref[0]) * d64 + b3_ref[0], 0.0) * d64
    o_ref[1] = jnp.maximum((a_ref[1] + h_ref[1]) * d64 + b3_ref[1], 0.0) * d64


def _k4(a_ref, x_ref, d64_ref, w4_ref, b4_ref, o_ref):
    i = pl.program_id(0)
    d64 = d64_ref[...]
    z0 = (a_ref[0] + x_ref[0]) * d64
    z1 = (a_ref[1] + x_ref[1]) * d64
    y4 = jnp.maximum(
        _dot(z0, w4_ref[0:64, :]) + _dot(z1, w4_ref[64:128, :]) + b4_ref[...],
        0.0)
    row = i * BN + lax.broadcasted_iota(jnp.int32, (BN, 1), 0)
    y4 = jnp.where(row < N, y4, 0.0)
    part = jnp.sum(y4, axis=0, keepdims=True) * (1.0 / N)

    @pl.when(i == 0)
    def _():
        o_ref[...] = part

    @pl.when(i > 0)
    def _():
        o_ref[...] = o_ref[...] + part


def _split_spec(dh):
    return pl.BlockSpec((2, BN, dh), lambda i: (0, i, 0))


def _col_spec(dh):
    return pl.BlockSpec((BN, dh), lambda i: (i, 0))


def _full_spec(shape):
    nd = len(shape)
    return pl.BlockSpec(shape, lambda i, _n=nd: (0,) * _n)


def _tc_call(body, in_specs, out_specs, out_shape, acc=False):
    sem = ("arbitrary",) if acc else ("parallel",)
    return pl.pallas_call(
        body,
        grid=(_GRID,),
        in_specs=in_specs,
        out_specs=out_specs,
        out_shape=out_shape,
        compiler_params=pltpu.CompilerParams(dimension_semantics=sem),
    )


def kernel(x, edge_index, edge_attr, W1, b1, W2, b2, W3, b3, W4, b4):
    del edge_attr
    src = edge_index[0]
    dst = edge_index[1]

    degp = _degree_counts(dst.reshape(NSC * NSUB, EPS))
    dinv2d = pl.pallas_call(
        _kdeg,
        out_shape=jax.ShapeDtypeStruct((NPAD // 128, 128), _F32),
    )(degp.reshape(NSC * NSUB, NPAD // 128, 128))
    dinv = dinv2d.reshape(NPAD)
    d64 = jnp.broadcast_to(dinv[:, None], (NPAD, 64))

    srcp = jnp.concatenate([src, jnp.zeros((EP - E,), src.dtype)])
    dstp = jnp.concatenate([dst, jnp.full((EP - E,), NPAD - 1, dst.dtype)])
    srcr2 = jnp.stack([srcp, srcp + NPAD]).reshape(NSC, NSUB, NIT, CH)
    dstr = dstp.reshape(NSUB, NIT, CH)
    x_pad = jnp.pad(x, ((0, NPAD - N), (0, 0)))
    x_split = x_pad.reshape(NPAD, 2, 64).transpose(1, 0, 2)
    b1r = b1.reshape(1, 512)
    w2q = W2.reshape(512, 4, 64).transpose(1, 0, 2)
    b2q = b2.reshape(4, 1, 64)
    w3q = W3.reshape(4, 64, 2, 64).transpose(0, 2, 1, 3)
    b3s = b3.reshape(2, 1, 64)
    b4r = b4.reshape(1, 200)
    sds = jax.ShapeDtypeStruct((2, NPAD, 64), _F32)

    xs1 = _tc_call(
        _k0,
        [_split_spec(64), _col_spec(64)],
        _split_spec(64),
        sds,
    )(x_split, d64)
    a1 = _aggregate(xs1, srcr2, dstr)
    h2a, h2b = _tc_call(
        _k12,
        [_split_spec(64), _split_spec(64), _col_spec(64),
         _full_spec((128, 512)), _full_spec((1, 512)), _full_spec((4, 512, 64))],
        [_split_spec(64), _split_spec(64)],
        [sds, sds],
    )(a1, xs1, d64, W1, b1r, w2q)

    a2a = _aggregate(h2a, srcr2, dstr)
    a2b = _aggregate(h2b, srcr2, dstr)
    h3 = _tc_call(
        _k23,
        [_split_spec(64), _split_spec(64), _split_spec(64), _split_spec(64),
         _col_spec(64), _full_spec((4, 1, 64)), _full_spec((4, 2, 64, 64))],
        _split_spec(64),
        sds,
    )(a2a, a2b, h2a, h2b, d64, b2q, w3q)

    a3 = _aggregate(h3, srcr2, dstr)
    xs4 = _tc_call(
        _k34,
        [_split_spec(64), _split_spec(64), _col_spec(64), _full_spec((2, 1, 64))],
        _split_spec(64),
        sds,
    )(a3, h3, d64, b3s)

    a4 = _aggregate(xs4, srcr2, dstr)
    out = _tc_call(
        _k4,
        [_split_spec(64), _split_spec(64), _col_spec(64),
         _full_spec((128, 200)), _full_spec((1, 200))],
        pl.BlockSpec((1, 200), lambda i: (0, 0)),
        jax.ShapeDtypeStruct((1, 200), _F32),
        acc=True,
    )(a4, xs4, d64, W4, b4r)
    return out


# speedup vs baseline: 2.2925x; 1.0001x over previous
import jax
import jax.numpy as jnp
from jax import lax
from jax.experimental import pallas as pl
from jax.experimental.pallas import tpu as pltpu
from jax.experimental.pallas import tpu_sc as plsc

N = 10000
E = 320000
NPAD = 10240
NSC = 2
NSUB = 16
CH = 128
EP = 327680
RPT = NPAD // NSUB
EPS = E // (NSC * NSUB)
NIT = EP // (NSUB * CH)
BN = 1024


def _sc_mesh():
    return plsc.VectorSubcoreMesh(core_axis_name="c", subcore_axis_name="s")


def _deg_body(dst_hbm, out_hbm, dbuf, part):
    c = lax.axis_index("c")
    s = lax.axis_index("s")
    wid = c * NSUB + s
    pltpu.sync_copy(dst_hbm.at[wid], dbuf)
    z16 = jnp.zeros((16,), jnp.float32)

    def zero_body(i, _):
        part[pl.ds(i * 16, 16)] = z16
        return 0

    lax.fori_loop(0, NPAD // 16, zero_body, 0)
    ones = jnp.ones((16,), jnp.float32)

    def cnt_body(i, _):
        iv = dbuf[pl.ds(i * 16, 16)]
        plsc.addupdate_scatter(part, [iv], ones)
        return 0

    lax.fori_loop(0, EPS // 16, cnt_body, 0)
    pltpu.sync_copy(part, out_hbm.at[wid])


def _degree_counts(dst_r):
    return pl.kernel(
        _deg_body,
        out_type=jax.ShapeDtypeStruct((NSC * NSUB, NPAD), jnp.float32),
        mesh=_sc_mesh(),
        compiler_params=pltpu.CompilerParams(needs_layout_passes=False),
        scratch_types=[
            pltpu.VMEM((EPS,), jnp.int32),
            pltpu.VMEM((NPAD,), jnp.float32),
        ],
    )(dst_r)


DH = 64


NB = 4
NG = NIT // NB


def _agg_body(xs_hbm, srcr2_hbm, dstr_hbm, out_hbm,
              didx, sidx, zbuf, acc, *ring):
    rows = ring[:NB]
    sems = ring[NB:]
    c = lax.axis_index("c")
    s = lax.axis_index("s")
    pltpu.sync_copy(srcr2_hbm.at[c, s], sidx)
    pltpu.sync_copy(dstr_hbm.at[s], didx)
    z16 = jnp.zeros((16,), jnp.float32)

    def zrow(i, _):
        def zcol(j, _):
            zbuf[i, pl.ds(j * 16, 16)] = z16
            return 0

        lax.fori_loop(0, DH // 16, zcol, 0)
        return 0

    lax.fori_loop(0, CH, zrow, 0)

    def zacc(k, _):
        pltpu.sync_copy(zbuf, acc.at[pl.ds(s * RPT + k * CH, CH)])
        return 0

    lax.fori_loop(0, RPT // CH, zacc, 0)
    plsc.subcore_barrier()

    for b in range(NB):
        pltpu.async_copy(xs_hbm.at[sidx.at[b]], rows[b], sems[b])

    def _slot(i, b):
        pltpu.make_async_copy(xs_hbm.at[sidx.at[i]], rows[b], sems[b]).wait()
        pltpu.sync_copy(rows[b], acc.at[didx.at[i]], add=True)

        @pl.when(i + NB < NIT)
        def _():
            pltpu.async_copy(xs_hbm.at[sidx.at[i + NB]], rows[b], sems[b])

    def grp(p, _):
        base = p * NB
        for b in range(NB):
            _slot(base + b, b)
        return 0

    lax.fori_loop(0, NG, grp, 0)
    plsc.subcore_barrier()
    pltpu.sync_copy(acc.at[pl.ds(s * RPT, RPT)],
                    out_hbm.at[pl.ds(c * NPAD + s * RPT, RPT)])


def _aggregate(xs, srcr2, dstr):
    return pl.kernel(
        _agg_body,
        out_type=jax.ShapeDtypeStruct((2 * NPAD, DH), jnp.float32),
        mesh=_sc_mesh(),
        compiler_params=pltpu.CompilerParams(
            needs_layout_passes=False, use_tc_tiling_on_sc=False),
        scratch_types=(
            [pltpu.VMEM((NIT, CH), jnp.int32),
             pltpu.VMEM((NIT, CH), jnp.int32),
             pltpu.VMEM((CH, DH), jnp.float32),
             pltpu.VMEM_SHARED((NPAD, DH), jnp.float32)]
            + [pltpu.VMEM((CH, DH), jnp.float32) for _ in range(NB)]
            + [pltpu.SemaphoreType.DMA for _ in range(NB)]
        ),
    )(xs.reshape(2 * NPAD, DH), srcr2, dstr).reshape(2, NPAD, DH)


_F32 = jnp.float32
_GRID = NPAD // BN


def _dot(a, b):
    return jnp.dot(a, b, preferred_element_type=_F32)


def _kdeg(p_ref, o_ref):
    o_ref[...] = lax.rsqrt(jnp.sum(p_ref[...], axis=0) + 1.0)


def _k0(x_ref, d_ref, o_ref):
    d = d_ref[...]
    o_ref[0] = x_ref[0] * d
    o_ref[1] = x_ref[1] * d


def _k12(a_ref, x_ref, d64_ref, w1_ref, b1_ref, w2_ref, oa_ref, ob_ref):
    d64 = d64_ref[...]
    z0 = (a_ref[0] + x_ref[0]) * d64
    z1 = (a_ref[1] + x_ref[1]) * d64
    y1 = jnp.maximum(
        _dot(z0, w1_ref[0:64, :]) + _dot(z1, w1_ref[64:128, :]) + b1_ref[...],
        0.0)
    oa_ref[0] = _dot(y1, w2_ref[0]) * d64
    oa_ref[1] = _dot(y1, w2_ref[1]) * d64
    ob_ref[0] = _dot(y1, w2_ref[2]) * d64
    ob_ref[1] = _dot(y1, w2_ref[3]) * d64


def _k23(aa_ref, ab_ref, ha_ref, hb_ref, d64_ref, b2_ref, w3_ref, o_ref):
    d64 = d64_ref[...]
    y2 = [
        jnp.maximum((aa_ref[0] + ha_ref[0]) * d64 + b2_ref[0], 0.0),
        jnp.maximum((aa_ref[1] + ha_ref[1]) * d64 + b2_ref[1], 0.0),
        jnp.maximum((ab_ref[0] + hb_ref[0]) * d64 + b2_ref[2], 0.0),
        jnp.maximum((ab_ref[1] + hb_ref[1]) * d64 + b2_ref[3], 0.0),
    ]
    for co in range(2):
        o_ref[co] = sum(_dot(y2[q], w3_ref[q, co]) for q in range(4)) * d64


def _k34(a_ref, h_ref, d64_ref, b3_ref, o_ref):
    d64 = d64_ref[...]
    o_ref[0] = jnp.maximum((a_ref[0] + h_ref[0]) * d64 + b3_ref[0], 0.0) * d64
    o_ref[1] = jnp.maximum((a_ref[1] + h_ref[1]) * d64 + b3_ref[1], 0.0) * d64


def _k4(a_ref, x_ref, d64_ref, w4_ref, b4_ref, o_ref):
    i = pl.program_id(0)
    d64 = d64_ref[...]
    z0 = (a_ref[0] + x_ref[0]) * d64
    z1 = (a_ref[1] + x_ref[1]) * d64
    y4 = jnp.maximum(
        _dot(z0, w4_ref[0:64, :]) + _dot(z1, w4_ref[64:128, :]) + b4_ref[...],
        0.0)
    row = i * BN + lax.broadcasted_iota(jnp.int32, (BN, 1), 0)
    y4 = jnp.where(row < N, y4, 0.0)
    part = jnp.sum(y4, axis=0, keepdims=True) * (1.0 / N)

    @pl.when(i == 0)
    def _():
        o_ref[...] = part

    @pl.when(i > 0)
    def _():
        o_ref[...] = o_ref[...] + part


def _split_spec(dh):
    return pl.BlockSpec((2, BN, dh), lambda i: (0, i, 0))


def _col_spec(dh):
    return pl.BlockSpec((BN, dh), lambda i: (i, 0))


def _full_spec(shape):
    nd = len(shape)
    return pl.BlockSpec(shape, lambda i, _n=nd: (0,) * _n)


def _tc_call(body, in_specs, out_specs, out_shape, acc=False):
    sem = ("arbitrary",) if acc else ("parallel",)
    return pl.pallas_call(
        body,
        grid=(_GRID,),
        in_specs=in_specs,
        out_specs=out_specs,
        out_shape=out_shape,
        compiler_params=pltpu.CompilerParams(dimension_semantics=sem),
    )


def kernel(x, edge_index, edge_attr, W1, b1, W2, b2, W3, b3, W4, b4):
    del edge_attr
    src = edge_index[0]
    dst = edge_index[1]

    degp = _degree_counts(dst.reshape(NSC * NSUB, EPS))
    dinv2d = pl.pallas_call(
        _kdeg,
        out_shape=jax.ShapeDtypeStruct((NPAD // 128, 128), _F32),
    )(degp.reshape(NSC * NSUB, NPAD // 128, 128))
    dinv = dinv2d.reshape(NPAD)
    d64 = jnp.broadcast_to(dinv[:, None], (NPAD, 64))

    srcp = jnp.concatenate([src, jnp.zeros((EP - E,), src.dtype)])
    dstp = jnp.concatenate([dst, jnp.full((EP - E,), NPAD - 1, dst.dtype)])
    srcr2 = jnp.stack([srcp, srcp + NPAD]).reshape(NSC, NSUB, NIT, CH)
    dstr = dstp.reshape(NSUB, NIT, CH)
    x_pad = jnp.pad(x, ((0, NPAD - N), (0, 0)))
    x_split = x_pad.reshape(NPAD, 2, 64).transpose(1, 0, 2)
    b1r = b1.reshape(1, 512)
    w2q = W2.reshape(512, 4, 64).transpose(1, 0, 2)
    b2q = b2.reshape(4, 1, 64)
    w3q = W3.reshape(4, 64, 2, 64).transpose(0, 2, 1, 3)
    b3s = b3.reshape(2, 1, 64)
    b4r = b4.reshape(1, 200)
    sds = jax.ShapeDtypeStruct((2, NPAD, 64), _F32)

    xs1 = _tc_call(
        _k0,
        [_split_spec(64), _col_spec(64)],
        _split_spec(64),
        sds,
    )(x_split, d64)
    a1 = _aggregate(xs1, srcr2, dstr)
    h2a, h2b = _tc_call(
        _k12,
        [_split_spec(64), _split_spec(64), _col_spec(64),
         _full_spec((128, 512)), _full_spec((1, 512)), _full_spec((4, 512, 64))],
        [_split_spec(64), _split_spec(64)],
        [sds, sds],
    )(a1, xs1, d64, W1, b1r, w2q)

    a2a = _aggregate(h2a, srcr2, dstr)
    a2b = _aggregate(h2b, srcr2, dstr)
    h3 = _tc_call(
        _k23,
        [_split_spec(64), _split_spec(64), _split_spec(64), _split_spec(64),
         _col_spec(64), _full_spec((4, 1, 64)), _full_spec((4, 2, 64, 64))],
        _split_spec(64),
        sds,
    )(a2a, a2b, h2a, h2b, d64, b2q, w3q)

    a3 = _aggregate(h3, srcr2, dstr)
    xs4 = _tc_call(
        _k34,
        [_split_spec(64), _split_spec(64), _col_spec(64), _full_spec((2, 1, 64))],
        _split_spec(64),
        sds,
    )(a3, h3, d64, b3s)

    a4 = _aggregate(xs4, srcr2, dstr)
    out = _tc_call(
        _k4,
        [_split_spec(64), _split_spec(64), _col_spec(64),
         _full_spec((128, 200)), _full_spec((1, 200))],
        pl.BlockSpec((1, 200), lambda i: (0, 0)),
        jax.ShapeDtypeStruct((1, 200), _F32),
        acc=True,
    )(a4, xs4, d64, W4, b4r)
    return out
